# Initial kernel scaffold; baseline (speedup 1.0000x reference)
import functools

import jax
import jax.numpy as jnp
from jax import lax
from jax.experimental import pallas as pl
from jax.experimental.pallas import tpu as pltpu
from jax.experimental.pallas import tpu_sc as plsc

N = 10000
E = 160000
HID = 64
H = 4
C = 64
HC = H * C
NB = 1000
GRID = N // NB
NSUB = 16
ES = E // NSUB
CHK = 128
NCHK = ES // CHK
TAIL = ES - NCHK * CHK
ROWS_PER_SUB = N // NSUB
F32 = jnp.float32
I32 = jnp.int32


def _ln(x, g, b):
    mu = jnp.mean(x, axis=-1, keepdims=True)
    var = jnp.mean((x - mu) ** 2, axis=-1, keepdims=True)
    return (x - mu) * jax.lax.rsqrt(var + 1e-5) * g + b


def _elu(x):
    return jnp.where(x > 0, x, jnp.expm1(jnp.minimum(x, 0.0)))


def _tc_pre_body(xs_ref, xr_ref, wxs_ref, bxs_ref, lng_ref, lnb_ref,
                 wxr_ref, bxr_ref, psd_ref, aa_ref, cv_ref, wx_ref,
                 z_ref, r_ref, tab_ref, xh_ref, smax_ref):
    z = jnp.dot(xs_ref[...], wxs_ref[...], preferred_element_type=F32) + bxs_ref[...]
    z = _elu(_ln(z, lng_ref[...], lnb_ref[...]))
    r = jnp.dot(xr_ref[...], wxr_ref[...], preferred_element_type=F32) + bxr_ref[...]
    sd = (jnp.dot(z, psd_ref[...], preferred_element_type=F32)
          + jnp.dot(r, aa_ref[...], preferred_element_type=F32) + cv_ref[...])
    xh = jnp.dot(z, wx_ref[...], preferred_element_type=F32)
    z_ref[...] = z
    r_ref[...] = r
    tab_ref[...] = sd
    xh_ref[0] = xh[:, :128]
    xh_ref[1] = xh[:, 128:]
    smax_ref[...] = jnp.max(sd, axis=0, keepdims=True)[None]


def _tc_pre(x_seq, x_residue, wxs, bxs, lng, lnb, wxr, bxr, psd, aa, cv, wx):
    full = lambda s: pl.BlockSpec(s, lambda i: tuple(0 for _ in s))
    return pl.pallas_call(
        _tc_pre_body,
        grid=(GRID,),
        in_specs=[
            pl.BlockSpec((NB, 128), lambda i: (i, 0)),
            pl.BlockSpec((NB, 128), lambda i: (i, 0)),
            full((128, HID)), full((1, HID)), full((1, HID)), full((1, HID)),
            full((128, HID)), full((1, HID)),
            full((HID, 8)), full((HID, 8)), full((1, 8)),
            full((HID, HC)),
        ],
        out_specs=[
            pl.BlockSpec((NB, HID), lambda i: (i, 0)),
            pl.BlockSpec((NB, HID), lambda i: (i, 0)),
            pl.BlockSpec((NB, 8), lambda i: (i, 0)),
            pl.BlockSpec((2, NB, 128), lambda i: (0, i, 0)),
            pl.BlockSpec((1, 1, 8), lambda i: (i, 0, 0)),
        ],
        out_shape=[
            jax.ShapeDtypeStruct((N, HID), F32),
            jax.ShapeDtypeStruct((N, HID), F32),
            jax.ShapeDtypeStruct((N, 8), F32),
            jax.ShapeDtypeStruct((2, N, 128), F32),
            jax.ShapeDtypeStruct((GRID, 1, 8), F32),
        ],
    )(x_seq, x_residue, wxs, bxs, lng, lnb, wxr, bxr, psd, aa, cv, wx)


def _tc_mid_body(last, agg_ref, s_ref, zp_ref, r_ref, wagg_ref, bias_ref,
                 lng_ref, lnb_ref, *rest):
    if last:
        wout_ref, bout_ref, y_ref = rest
    else:
        psd_ref, aa_ref, cv_ref, wx_ref, z_ref, tab_ref, xh_ref, smax_ref = rest
    num = jnp.concatenate([agg_ref[0], agg_ref[1]], axis=-1)
    den = jnp.concatenate([s_ref[0][:, 0:2], s_ref[1][:, 0:2]], axis=-1)
    deninv = 1.0 / (den + 1e-16)
    normed = (num.reshape(NB, H, C) * deninv[:, :, None]).reshape(NB, HC)
    out = jnp.dot(normed, wagg_ref[...], preferred_element_type=F32) + bias_ref[...]
    z = _ln(out + zp_ref[...], lng_ref[...], lnb_ref[...])
    if last:
        y_ref[...] = jnp.dot(z, wout_ref[...], preferred_element_type=F32) + bout_ref[...]
        return
    r = r_ref[...]
    sd = (jnp.dot(z, psd_ref[...], preferred_element_type=F32)
          + jnp.dot(r, aa_ref[...], preferred_element_type=F32) + cv_ref[...])
    xh = jnp.dot(z, wx_ref[...], preferred_element_type=F32)
    z_ref[...] = z
    tab_ref[...] = sd
    xh_ref[0] = xh[:, :128]
    xh_ref[1] = xh[:, 128:]
    smax_ref[...] = jnp.max(sd, axis=0, keepdims=True)[None]


def _tc_mid(last, agg, s, zp, r, wagg, bias, lng, lnb, *rest):
    full = lambda s_: pl.BlockSpec(s_, lambda i: tuple(0 for _ in s_))
    in_specs = [
        pl.BlockSpec((2, NB, 128), lambda i: (0, i, 0)),
        pl.BlockSpec((2, NB, 16), lambda i: (0, i, 0)),
        pl.BlockSpec((NB, HID), lambda i: (i, 0)),
        pl.BlockSpec((NB, HID), lambda i: (i, 0)),
        full((HC, HID)), full((1, HID)), full((1, HID)), full((1, HID)),
    ]
    if last:
        in_specs += [full((HID, HID)), full((1, HID))]
        out_specs = [pl.BlockSpec((NB, HID), lambda i: (i, 0))]
        out_shape = [jax.ShapeDtypeStruct((N, HID), F32)]
    else:
        in_specs += [full((HID, 8)), full((HID, 8)), full((1, 8)), full((HID, HC))]
        out_specs = [
            pl.BlockSpec((NB, HID), lambda i: (i, 0)),
            pl.BlockSpec((NB, 8), lambda i: (i, 0)),
            pl.BlockSpec((2, NB, 128), lambda i: (0, i, 0)),
            pl.BlockSpec((1, 1, 8), lambda i: (i, 0, 0)),
        ]
        out_shape = [
            jax.ShapeDtypeStruct((N, HID), F32),
            jax.ShapeDtypeStruct((N, 8), F32),
            jax.ShapeDtypeStruct((2, N, 128), F32),
            jax.ShapeDtypeStruct((GRID, 1, 8), F32),
        ]
    return pl.pallas_call(
        functools.partial(_tc_mid_body, last),
        grid=(GRID,),
        in_specs=in_specs,
        out_specs=out_specs,
        out_shape=out_shape,
    )(agg, s, zp, r, wagg, bias, lng, lnb, *rest)


def _sc_body(tab_hbm, xh_hbm, src_hbm, dst_hbm, gmax_hbm,
             agg_out, s_out,
             idxs, idxd, idxg, srows, drows, xbuf, exbuf,
             idxs_t, idxd_t, idxg_t, srows_t, drows_t, xbuf_t, exbuf_t,
             gvec, zb, zbs, agg_sh, s_sh, sem):
    c = lax.axis_index("c")
    sid = lax.axis_index("s")
    zero16 = jnp.zeros((16,), F32)
    iota16 = lax.iota(I32, 16)

    pltpu.sync_copy(gmax_hbm, gvec)

    @pl.loop(0, 125)
    def _zb(i):
        for j in range(8):
            zb[i, pl.ds(j * 16, 16)] = zero16
        zbs[i, :] = zero16

    @pl.loop(0, CHK)
    def _ze(i):
        exbuf[i, :] = zero16

    @pl.loop(0, 16)
    def _zet(i):
        exbuf_t[i, :] = zero16

    for k in range(5):
        off = sid * ROWS_PER_SUB + k * 125
        pltpu.sync_copy(zb, agg_sh.at[pl.ds(off, 125)])
        pltpu.sync_copy(zbs, s_sh.at[pl.ds(off, 125)])
    plsc.subcore_barrier()

    gbase = jnp.where(c == 0, 0, N).astype(I32)
    g0 = jnp.where(c == 0, gvec[0], gvec[2])
    g1 = jnp.where(c == 0, gvec[1], gvec[3])
    cf = c * 2

    def lrelu(x):
        return jnp.maximum(x, x * 0.2)

    def do_chunk(ebase, chn, b_idxs, b_idxd, b_idxg, b_srows, b_drows,
                 b_xbuf, b_exbuf):
        pltpu.sync_copy(src_hbm.at[pl.ds(ebase, chn)], b_idxs)
        pltpu.sync_copy(dst_hbm.at[pl.ds(ebase, chn)], b_idxd)

        @pl.loop(0, chn // 16)
        def _adj(i):
            b_idxg[pl.ds(i * 16, 16)] = b_idxs[pl.ds(i * 16, 16)] + gbase

        cp1 = pltpu.async_copy(tab_hbm.at[b_idxs], b_srows, sem)
        cp2 = pltpu.async_copy(tab_hbm.at[b_idxd], b_drows, sem)
        cp3 = pltpu.async_copy(xh_hbm.at[b_idxg], b_xbuf, sem)
        cp1.wait()
        cp2.wait()
        cp3.wait()

        @pl.loop(0, chn // 16)
        def _grp(gi):
            rows = gi * 16 + iota16
            cS0 = jnp.full((16,), cf, I32)
            cS1 = jnp.full((16,), cf + 1, I32)
            cD0 = jnp.full((16,), cf + 4, I32)
            cD1 = jnp.full((16,), cf + 5, I32)
            s0 = plsc.load_gather(b_srows, [rows, cS0])
            s1 = plsc.load_gather(b_srows, [rows, cS1])
            d0 = plsc.load_gather(b_drows, [rows, cD0])
            d1 = plsc.load_gather(b_drows, [rows, cD1])
            ex0 = jnp.exp(lrelu(s0 + d0) - lrelu(d0 + g0))
            ex1 = jnp.exp(lrelu(s1 + d1) - lrelu(d1 + g1))
            plsc.store_scatter(b_exbuf, [rows, jnp.zeros((16,), I32)], ex0)
            plsc.store_scatter(b_exbuf, [rows, jnp.ones((16,), I32)], ex1)
            for f in range(128):
                colv = jnp.full((16,), f, I32)
                v = plsc.load_gather(b_xbuf, [rows, colv])
                v = v * (ex0 if f < 64 else ex1)
                plsc.store_scatter(b_xbuf, [rows, colv], v)

        pltpu.sync_copy(b_exbuf, s_sh.at[b_idxd], add=True)
        pltpu.sync_copy(b_xbuf, agg_sh.at[b_idxd], add=True)

    @pl.loop(0, NCHK)
    def _chunks(k):
        do_chunk(sid * ES + k * CHK, CHK,
                 idxs, idxd, idxg, srows, drows, xbuf, exbuf)

    if TAIL:
        do_chunk(sid * ES + NCHK * CHK, TAIL,
                 idxs_t, idxd_t, idxg_t, srows_t, drows_t, xbuf_t, exbuf_t)

    plsc.subcore_barrier()

    rowoff = sid * ROWS_PER_SUB
    outoff = c * N + rowoff
    pltpu.sync_copy(agg_sh.at[pl.ds(rowoff, ROWS_PER_SUB)],
                    agg_out.at[pl.ds(outoff, ROWS_PER_SUB)])
    pltpu.sync_copy(s_sh.at[pl.ds(rowoff, ROWS_PER_SUB)],
                    s_out.at[pl.ds(outoff, ROWS_PER_SUB)])


_sc_edge = pl.kernel(
    _sc_body,
    out_type=(
        jax.ShapeDtypeStruct((2 * N, 128), F32),
        jax.ShapeDtypeStruct((2 * N, 16), F32),
    ),
    mesh=plsc.VectorSubcoreMesh(core_axis_name="c", subcore_axis_name="s"),
    scratch_types=[
        pltpu.VMEM((CHK,), I32),
        pltpu.VMEM((CHK,), I32),
        pltpu.VMEM((CHK,), I32),
        pltpu.VMEM((CHK, 8), F32),
        pltpu.VMEM((CHK, 8), F32),
        pltpu.VMEM((CHK, 128), F32),
        pltpu.VMEM((CHK, 16), F32),
        pltpu.VMEM((16,), I32),
        pltpu.VMEM((16,), I32),
        pltpu.VMEM((16,), I32),
        pltpu.VMEM((16, 8), F32),
        pltpu.VMEM((16, 8), F32),
        pltpu.VMEM((16, 128), F32),
        pltpu.VMEM((16, 16), F32),
        pltpu.VMEM((16,), F32),
        pltpu.VMEM((125, 128), F32),
        pltpu.VMEM((125, 16), F32),
        pltpu.VMEM_SHARED((N, 128), F32),
        pltpu.VMEM_SHARED((N, 16), F32),
        pltpu.SemaphoreType.DMA,
    ],
)


def _fold_layer(p, wg, b_rd):
    att_src = p['att_src'][0]
    att_dst = p['att_dst'][0]
    att_rd = p['att_rd'][0]
    v = jnp.einsum('khc,hc->kh', p['W_rd'].reshape(HID, H, C), att_rd)
    a = wg @ v
    c0 = b_rd @ v
    psrc = jnp.einsum('khc,hc->kh', p['W_x'].reshape(HID, H, C), att_src)
    pdst = jnp.einsum('khc,hc->kh', p['W_x'].reshape(HID, H, C), att_dst)
    psd = jnp.concatenate([psrc, pdst], axis=1)
    aa = jnp.concatenate([-a, a], axis=1)
    cv = jnp.concatenate([jnp.zeros((4,), F32), c0])[None]
    return psd, aa, cv


def _gmax16(smax):
    g = jnp.max(smax[:, 0, :4], axis=0)
    return jnp.concatenate([g, jnp.zeros((12,), F32)])


def kernel(x_seq, x_residue, edge_index, params):
    src = edge_index[0].astype(I32)
    dst = edge_index[1].astype(I32)
    row = lambda b: b[None]
    psd0, aa0, cv0 = _fold_layer(params['convs'][0], params['W_rd'], params['b_rd'])
    psd1, aa1, cv1 = _fold_layer(params['convs'][1], params['W_rd'], params['b_rd'])

    z0, r, tab0, xh0, smax0 = _tc_pre(
        x_seq, x_residue,
        params['W_xs'], row(params['b_xs']),
        row(params['ln_g'][0]), row(params['ln_b'][0]),
        params['W_xr'], row(params['b_xr']),
        psd0, aa0, cv0, params['convs'][0]['W_x'])

    agg0, s0 = _sc_edge(tab0, xh0.reshape(2 * N, 128), src, dst, _gmax16(smax0))

    p0 = params['convs'][0]
    z1, tab1, xh1, smax1 = _tc_mid(
        False, agg0.reshape(2, N, 128), s0.reshape(2, N, 16), z0, r,
        p0['W_agg'], row(p0['bias']),
        row(params['ln_g'][1]), row(params['ln_b'][1]),
        psd1, aa1, cv1, params['convs'][1]['W_x'])

    agg1, s1 = _sc_edge(tab1, xh1.reshape(2 * N, 128), src, dst, _gmax16(smax1))

    p1 = params['convs'][1]
    (y,) = _tc_mid(
        True, agg1.reshape(2, N, 128), s1.reshape(2, N, 16), z1, r,
        p1['W_agg'], row(p1['bias']),
        row(params['ln_g'][2]), row(params['ln_b'][2]),
        params['W_out'], row(params['b_out']))
    return y


# speedup vs baseline: 7.2242x; 7.2242x over previous
import functools

import jax
import jax.numpy as jnp
from jax import lax
from jax.experimental import pallas as pl
from jax.experimental.pallas import tpu as pltpu
from jax.experimental.pallas import tpu_sc as plsc

N = 10000
E = 160000
HID = 64
H = 4
C = 64
HC = H * C
NB = 1000
GRID = N // NB
NSUB = 16
ES = E // NSUB
CHK = 128
NCHK = ES // CHK
TAIL = ES - NCHK * CHK
NP_ = 10240
ROWS_PER_SUB = NP_ // NSUB
F32 = jnp.float32
I32 = jnp.int32


def _ln(x, g, b):
    mu = jnp.mean(x, axis=-1, keepdims=True)
    var = jnp.mean((x - mu) ** 2, axis=-1, keepdims=True)
    return (x - mu) * jax.lax.rsqrt(var + 1e-5) * g + b


def _elu(x):
    return jnp.where(x > 0, x, jnp.exp(jnp.minimum(x, 0.0)) - 1.0)


def _tc_pre_body(xs_ref, xr_ref, wxs_ref, bxs_ref, lng_ref, lnb_ref,
                 wxr_ref, bxr_ref, psd_ref, aa_ref, cv_ref, wx_ref,
                 z_ref, r_ref, tab_ref, xh_ref, smax_ref):
    z = jnp.dot(xs_ref[...], wxs_ref[...], preferred_element_type=F32) + bxs_ref[...]
    z = _elu(_ln(z, lng_ref[...], lnb_ref[...]))
    r = jnp.dot(xr_ref[...], wxr_ref[...], preferred_element_type=F32) + bxr_ref[...]
    sd = (jnp.dot(z, psd_ref[...], preferred_element_type=F32)
          + jnp.dot(r, aa_ref[...], preferred_element_type=F32) + cv_ref[...])
    xh = jnp.dot(z, wx_ref[...], preferred_element_type=F32)
    z_ref[...] = z
    r_ref[...] = r
    tab_ref[...] = sd
    xh_ref[0] = xh[:, :128]
    xh_ref[1] = xh[:, 128:]
    smax_ref[...] = jnp.max(sd, axis=0, keepdims=True)[None]


def _tc_pre(x_seq, x_residue, wxs, bxs, lng, lnb, wxr, bxr, psd, aa, cv, wx):
    full = lambda s: pl.BlockSpec(s, lambda i: tuple(0 for _ in s))
    return pl.pallas_call(
        _tc_pre_body,
        grid=(GRID,),
        in_specs=[
            pl.BlockSpec((NB, 128), lambda i: (i, 0)),
            pl.BlockSpec((NB, 128), lambda i: (i, 0)),
            full((128, HID)), full((1, HID)), full((1, HID)), full((1, HID)),
            full((128, HID)), full((1, HID)),
            full((HID, 8)), full((HID, 8)), full((1, 8)),
            full((HID, HC)),
        ],
        out_specs=[
            pl.BlockSpec((NB, HID), lambda i: (i, 0)),
            pl.BlockSpec((NB, HID), lambda i: (i, 0)),
            pl.BlockSpec((NB, 8), lambda i: (i, 0)),
            pl.BlockSpec((2, NB, 128), lambda i: (0, i, 0)),
            pl.BlockSpec((1, 1, 8), lambda i: (i, 0, 0)),
        ],
        out_shape=[
            jax.ShapeDtypeStruct((N, HID), F32),
            jax.ShapeDtypeStruct((N, HID), F32),
            jax.ShapeDtypeStruct((N, 8), F32),
            jax.ShapeDtypeStruct((2, N, 128), F32),
            jax.ShapeDtypeStruct((GRID, 1, 8), F32),
        ],
    )(x_seq, x_residue, wxs, bxs, lng, lnb, wxr, bxr, psd, aa, cv, wx)


def _tc_mid_body(last, agg_ref, s_ref, zp_ref, r_ref, wagg_ref, bias_ref,
                 lng_ref, lnb_ref, *rest):
    if last:
        wout_ref, bout_ref, y_ref = rest
    else:
        psd_ref, aa_ref, cv_ref, wx_ref, z_ref, tab_ref, xh_ref, smax_ref = rest
    num = jnp.concatenate([agg_ref[0], agg_ref[1]], axis=-1)
    den = jnp.concatenate([s_ref[0], s_ref[1]], axis=-1)
    deninv = 1.0 / (den + 1e-16)
    normed = (num.reshape(NB, H, C) * deninv[:, :, None]).reshape(NB, HC)
    out = jnp.dot(normed, wagg_ref[...], preferred_element_type=F32) + bias_ref[...]
    z = _ln(out + zp_ref[...], lng_ref[...], lnb_ref[...])
    if last:
        y_ref[...] = jnp.dot(z, wout_ref[...], preferred_element_type=F32) + bout_ref[...]
        return
    r = r_ref[...]
    sd = (jnp.dot(z, psd_ref[...], preferred_element_type=F32)
          + jnp.dot(r, aa_ref[...], preferred_element_type=F32) + cv_ref[...])
    xh = jnp.dot(z, wx_ref[...], preferred_element_type=F32)
    z_ref[...] = z
    tab_ref[...] = sd
    xh_ref[0] = xh[:, :128]
    xh_ref[1] = xh[:, 128:]
    smax_ref[...] = jnp.max(sd, axis=0, keepdims=True)[None]


def _tc_mid(last, agg, s, zp, r, wagg, bias, lng, lnb, *rest):
    full = lambda s_: pl.BlockSpec(s_, lambda i: tuple(0 for _ in s_))
    in_specs = [
        pl.BlockSpec((2, NB, 128), lambda i: (0, i, 0)),
        pl.BlockSpec((2, NB, 2), lambda i: (0, i, 0)),
        pl.BlockSpec((NB, HID), lambda i: (i, 0)),
        pl.BlockSpec((NB, HID), lambda i: (i, 0)),
        full((HC, HID)), full((1, HID)), full((1, HID)), full((1, HID)),
    ]
    if last:
        in_specs += [full((HID, HID)), full((1, HID))]
        out_specs = [pl.BlockSpec((NB, HID), lambda i: (i, 0))]
        out_shape = [jax.ShapeDtypeStruct((N, HID), F32)]
    else:
        in_specs += [full((HID, 8)), full((HID, 8)), full((1, 8)), full((HID, HC))]
        out_specs = [
            pl.BlockSpec((NB, HID), lambda i: (i, 0)),
            pl.BlockSpec((NB, 8), lambda i: (i, 0)),
            pl.BlockSpec((2, NB, 128), lambda i: (0, i, 0)),
            pl.BlockSpec((1, 1, 8), lambda i: (i, 0, 0)),
        ]
        out_shape = [
            jax.ShapeDtypeStruct((N, HID), F32),
            jax.ShapeDtypeStruct((N, 8), F32),
            jax.ShapeDtypeStruct((2, N, 128), F32),
            jax.ShapeDtypeStruct((GRID, 1, 8), F32),
        ]
    return pl.pallas_call(
        functools.partial(_tc_mid_body, last),
        grid=(GRID,),
        in_specs=in_specs,
        out_specs=out_specs,
        out_shape=out_shape,
    )(agg, s, zp, r, wagg, bias, lng, lnb, *rest)


def _sc_body(tab_hbm, xh_hbm, src_hbm, dst_hbm, gmax_hbm,
             agg_out, s_out,
             idxs, idxd, idxg, idxe0, idxe1, srows, drows, xbuf, exb0, exb1,
             idxs_t, idxd_t, idxg_t, idxe0_t, idxe1_t, srows_t, drows_t,
             xbuf_t, exb0_t, exb1_t,
             gvec, zb, zbs, agg_sh, s_sh, sem):
    c = lax.axis_index("c")
    sid = lax.axis_index("s")
    zero16 = jnp.zeros((16,), F32)
    iota16 = lax.iota(I32, 16)

    pltpu.sync_copy(gmax_hbm, gvec)

    @pl.loop(0, 128)
    def _zb(i):
        for j in range(8):
            zb[i, pl.ds(j * 16, 16)] = zero16

    @pl.loop(0, 2 * ROWS_PER_SUB // 16)
    def _zbs(i):
        zbs[pl.ds(i * 16, 16)] = zero16

    for k in range(5):
        off = sid * ROWS_PER_SUB + k * 128
        pltpu.sync_copy(zb, agg_sh.at[pl.ds(off, 128)])
    pltpu.sync_copy(zbs, s_sh.at[pl.ds(sid * 2 * ROWS_PER_SUB, 2 * ROWS_PER_SUB)])
    plsc.subcore_barrier()

    gbase = jnp.where(c == 0, 0, N).astype(I32)
    gv = gvec[...]
    g0 = jnp.where(c == 0, gv[0], gv[2])
    g1 = jnp.where(c == 0, gv[1], gv[3])
    cf = c * 2

    def lrelu(x):
        return jnp.maximum(x, x * 0.2)

    def do_chunk(ebase, chn, b_idxs, b_idxd, b_idxg, b_idxe0, b_idxe1,
                 b_srows, b_drows, b_xbuf, b_exb0, b_exb1):
        pltpu.sync_copy(src_hbm.at[pl.ds(ebase, chn)], b_idxs)
        pltpu.sync_copy(dst_hbm.at[pl.ds(ebase, chn)], b_idxd)

        @pl.loop(0, chn // 16)
        def _adj(i):
            sl = pl.ds(i * 16, 16)
            b_idxg[sl] = b_idxs[sl] + gbase
            d2 = b_idxd[sl] * 2
            b_idxe0[sl] = d2
            b_idxe1[sl] = d2 + 1

        cp1 = pltpu.async_copy(tab_hbm.at[b_idxs], b_srows, sem)
        cp2 = pltpu.async_copy(tab_hbm.at[b_idxd], b_drows, sem)
        cp3 = pltpu.async_copy(xh_hbm.at[b_idxg], b_xbuf, sem)
        cp1.wait()
        cp2.wait()
        cp3.wait()

        @pl.loop(0, chn // 16)
        def _grp(gi):
            rows = gi * 16 + iota16
            s0 = plsc.load_gather(b_srows, [rows, jnp.full((16,), cf, I32)])
            s1 = plsc.load_gather(b_srows, [rows, jnp.full((16,), cf + 1, I32)])
            d0 = plsc.load_gather(b_drows, [rows, jnp.full((16,), cf + 4, I32)])
            d1 = plsc.load_gather(b_drows, [rows, jnp.full((16,), cf + 5, I32)])
            ex0 = jnp.exp(lrelu(s0 + d0) - lrelu(d0 + g0))
            ex1 = jnp.exp(lrelu(s1 + d1) - lrelu(d1 + g1))
            esl = pl.ds(gi * 16, 16)
            b_exb0[esl] = ex0
            b_exb1[esl] = ex1
            for f in range(128):
                colv = jnp.full((16,), f, I32)
                v = plsc.load_gather(b_xbuf, [rows, colv])
                v = v * (ex0 if f < 64 else ex1)
                plsc.store_scatter(b_xbuf, [rows, colv], v)

        pltpu.sync_copy(b_exb0, s_sh.at[b_idxe0], add=True)
        pltpu.sync_copy(b_exb1, s_sh.at[b_idxe1], add=True)
        pltpu.sync_copy(b_xbuf, agg_sh.at[b_idxd], add=True)

    @pl.loop(0, NCHK)
    def _chunks(k):
        do_chunk(sid * ES + k * CHK, CHK,
                 idxs, idxd, idxg, idxe0, idxe1, srows, drows, xbuf, exb0, exb1)

    if TAIL:
        do_chunk(sid * ES + NCHK * CHK, TAIL,
                 idxs_t, idxd_t, idxg_t, idxe0_t, idxe1_t, srows_t, drows_t,
                 xbuf_t, exb0_t, exb1_t)

    plsc.subcore_barrier()

    rowoff = sid * ROWS_PER_SUB
    for k in range(5):
        off = rowoff + k * 128
        pltpu.sync_copy(agg_sh.at[pl.ds(off, 128)], zb)
        pltpu.sync_copy(zb, agg_out.at[pl.ds(c * NP_ + off, 128)])
    soff = sid * 2 * ROWS_PER_SUB
    pltpu.sync_copy(s_sh.at[pl.ds(soff, 2 * ROWS_PER_SUB)], zbs)
    pltpu.sync_copy(zbs, s_out.at[pl.ds(c * 2 * NP_ + soff, 2 * ROWS_PER_SUB)])


_sc_edge = pl.kernel(
    _sc_body,
    out_type=(
        jax.ShapeDtypeStruct((2 * NP_, 128), F32),
        jax.ShapeDtypeStruct((2 * 2 * NP_,), F32),
    ),
    mesh=plsc.VectorSubcoreMesh(core_axis_name="c", subcore_axis_name="s"),
    compiler_params=pltpu.CompilerParams(needs_layout_passes=False,
                                         use_tc_tiling_on_sc=False),
    scratch_types=[
        pltpu.VMEM((CHK,), I32),
        pltpu.VMEM((CHK,), I32),
        pltpu.VMEM((CHK,), I32),
        pltpu.VMEM((CHK,), I32),
        pltpu.VMEM((CHK,), I32),
        pltpu.VMEM((CHK, 8), F32),
        pltpu.VMEM((CHK, 8), F32),
        pltpu.VMEM((CHK, 128), F32),
        pltpu.VMEM((CHK,), F32),
        pltpu.VMEM((CHK,), F32),
        pltpu.VMEM((16,), I32),
        pltpu.VMEM((16,), I32),
        pltpu.VMEM((16,), I32),
        pltpu.VMEM((16,), I32),
        pltpu.VMEM((16,), I32),
        pltpu.VMEM((16, 8), F32),
        pltpu.VMEM((16, 8), F32),
        pltpu.VMEM((16, 128), F32),
        pltpu.VMEM((16,), F32),
        pltpu.VMEM((16,), F32),
        pltpu.VMEM((16,), F32),
        pltpu.VMEM((128, 128), F32),
        pltpu.VMEM((2 * ROWS_PER_SUB,), F32),
        pltpu.VMEM_SHARED((NP_, 128), F32),
        pltpu.VMEM_SHARED((2 * NP_,), F32),
        pltpu.SemaphoreType.DMA,
    ],
)


def _fold_layer(p, wg, b_rd):
    att_src = p['att_src'][0]
    att_dst = p['att_dst'][0]
    att_rd = p['att_rd'][0]
    v = jnp.einsum('khc,hc->kh', p['W_rd'].reshape(HID, H, C), att_rd)
    a = wg @ v
    c0 = b_rd @ v
    psrc = jnp.einsum('khc,hc->kh', p['W_x'].reshape(HID, H, C), att_src)
    pdst = jnp.einsum('khc,hc->kh', p['W_x'].reshape(HID, H, C), att_dst)
    psd = jnp.concatenate([psrc, pdst], axis=1)
    aa = jnp.concatenate([-a, a], axis=1)
    cv = jnp.concatenate([jnp.zeros((4,), F32), c0])[None]
    return psd, aa, cv


def _gmax16(smax):
    g = jnp.max(smax[:, 0, :4], axis=0)
    return jnp.concatenate([g, jnp.zeros((12,), F32)])


def kernel(x_seq, x_residue, edge_index, params):
    src = edge_index[0].astype(I32)
    dst = edge_index[1].astype(I32)
    row = lambda b: b[None]
    psd0, aa0, cv0 = _fold_layer(params['convs'][0], params['W_rd'], params['b_rd'])
    psd1, aa1, cv1 = _fold_layer(params['convs'][1], params['W_rd'], params['b_rd'])

    z0, r, tab0, xh0, smax0 = _tc_pre(
        x_seq, x_residue,
        params['W_xs'], row(params['b_xs']),
        row(params['ln_g'][0]), row(params['ln_b'][0]),
        params['W_xr'], row(params['b_xr']),
        psd0, aa0, cv0, params['convs'][0]['W_x'])

    agg0, s0 = _sc_edge(tab0, xh0.reshape(2 * N, 128), src, dst, _gmax16(smax0))

    p0 = params['convs'][0]
    z1, tab1, xh1, smax1 = _tc_mid(
        False, agg0.reshape(2, NP_, 128)[:, :N], s0.reshape(2, NP_, 2)[:, :N], z0, r,
        p0['W_agg'], row(p0['bias']),
        row(params['ln_g'][1]), row(params['ln_b'][1]),
        psd1, aa1, cv1, params['convs'][1]['W_x'])

    agg1, s1 = _sc_edge(tab1, xh1.reshape(2 * N, 128), src, dst, _gmax16(smax1))

    p1 = params['convs'][1]
    (y,) = _tc_mid(
        True, agg1.reshape(2, NP_, 128)[:, :N], s1.reshape(2, NP_, 2)[:, :N], z1, r,
        p1['W_agg'], row(p1['bias']),
        row(params['ln_g'][2]), row(params['ln_b'][2]),
        params['W_out'], row(params['b_out']))
    return y


# speedup vs baseline: 7.6138x; 1.0539x over previous
import functools

import jax
import jax.numpy as jnp
from jax import lax
from jax.experimental import pallas as pl
from jax.experimental.pallas import tpu as pltpu
from jax.experimental.pallas import tpu_sc as plsc

N = 10000
E = 160000
HID = 64
H = 4
C = 64
HC = H * C
NB = 1000
GRID = N // NB
NSUB = 16
ES = E // NSUB
CHK = 96
NCHK = ES // CHK
TAIL = ES - NCHK * CHK
NP_ = 10240
ROWS_PER_SUB = NP_ // NSUB
F32 = jnp.float32
I32 = jnp.int32


def _ln(x, g, b):
    mu = jnp.mean(x, axis=-1, keepdims=True)
    var = jnp.mean((x - mu) ** 2, axis=-1, keepdims=True)
    return (x - mu) * jax.lax.rsqrt(var + 1e-5) * g + b


def _elu(x):
    return jnp.where(x > 0, x, jnp.exp(jnp.minimum(x, 0.0)) - 1.0)


def _tc_pre_body(xs_ref, xr_ref, wxs_ref, bxs_ref, lng_ref, lnb_ref,
                 wxr_ref, bxr_ref, psd_ref, aa_ref, cv_ref, wx_ref,
                 z_ref, r_ref, tab_ref, xh_ref, smax_ref):
    z = jnp.dot(xs_ref[...], wxs_ref[...], preferred_element_type=F32) + bxs_ref[...]
    z = _elu(_ln(z, lng_ref[...], lnb_ref[...]))
    r = jnp.dot(xr_ref[...], wxr_ref[...], preferred_element_type=F32) + bxr_ref[...]
    sd = (jnp.dot(z, psd_ref[...], preferred_element_type=F32)
          + jnp.dot(r, aa_ref[...], preferred_element_type=F32) + cv_ref[...])
    xh = jnp.dot(z, wx_ref[...], preferred_element_type=F32)
    z_ref[...] = z
    r_ref[...] = r
    tab_ref[...] = sd
    xh_ref[0] = xh[:, :128]
    xh_ref[1] = xh[:, 128:]
    smax_ref[...] = jnp.max(sd, axis=0, keepdims=True)[None]


def _tc_pre(x_seq, x_residue, wxs, bxs, lng, lnb, wxr, bxr, psd, aa, cv, wx):
    full = lambda s: pl.BlockSpec(s, lambda i: tuple(0 for _ in s))
    return pl.pallas_call(
        _tc_pre_body,
        grid=(GRID,),
        in_specs=[
            pl.BlockSpec((NB, 128), lambda i: (i, 0)),
            pl.BlockSpec((NB, 128), lambda i: (i, 0)),
            full((128, HID)), full((1, HID)), full((1, HID)), full((1, HID)),
            full((128, HID)), full((1, HID)),
            full((HID, 8)), full((HID, 8)), full((1, 8)),
            full((HID, HC)),
        ],
        out_specs=[
            pl.BlockSpec((NB, HID), lambda i: (i, 0)),
            pl.BlockSpec((NB, HID), lambda i: (i, 0)),
            pl.BlockSpec((NB, 8), lambda i: (i, 0)),
            pl.BlockSpec((2, NB, 128), lambda i: (0, i, 0)),
            pl.BlockSpec((1, 1, 8), lambda i: (i, 0, 0)),
        ],
        out_shape=[
            jax.ShapeDtypeStruct((N, HID), F32),
            jax.ShapeDtypeStruct((N, HID), F32),
            jax.ShapeDtypeStruct((N, 8), F32),
            jax.ShapeDtypeStruct((2, N, 128), F32),
            jax.ShapeDtypeStruct((GRID, 1, 8), F32),
        ],
    )(x_seq, x_residue, wxs, bxs, lng, lnb, wxr, bxr, psd, aa, cv, wx)


def _tc_mid_body(last, agg_ref, s_ref, zp_ref, r_ref, wagg_ref, bias_ref,
                 lng_ref, lnb_ref, *rest):
    if last:
        wout_ref, bout_ref, y_ref = rest
    else:
        psd_ref, aa_ref, cv_ref, wx_ref, z_ref, tab_ref, xh_ref, smax_ref = rest
    num = jnp.concatenate([agg_ref[0], agg_ref[1]], axis=-1)
    den = jnp.concatenate([s_ref[0], s_ref[1]], axis=-1)
    deninv = 1.0 / (den + 1e-16)
    normed = (num.reshape(NB, H, C) * deninv[:, :, None]).reshape(NB, HC)
    out = jnp.dot(normed, wagg_ref[...], preferred_element_type=F32) + bias_ref[...]
    z = _ln(out + zp_ref[...], lng_ref[...], lnb_ref[...])
    if last:
        y_ref[...] = jnp.dot(z, wout_ref[...], preferred_element_type=F32) + bout_ref[...]
        return
    r = r_ref[...]
    sd = (jnp.dot(z, psd_ref[...], preferred_element_type=F32)
          + jnp.dot(r, aa_ref[...], preferred_element_type=F32) + cv_ref[...])
    xh = jnp.dot(z, wx_ref[...], preferred_element_type=F32)
    z_ref[...] = z
    tab_ref[...] = sd
    xh_ref[0] = xh[:, :128]
    xh_ref[1] = xh[:, 128:]
    smax_ref[...] = jnp.max(sd, axis=0, keepdims=True)[None]


def _tc_mid(last, agg, s, zp, r, wagg, bias, lng, lnb, *rest):
    full = lambda s_: pl.BlockSpec(s_, lambda i: tuple(0 for _ in s_))
    in_specs = [
        pl.BlockSpec((2, NB, 128), lambda i: (0, i, 0)),
        pl.BlockSpec((2, NB, 2), lambda i: (0, i, 0)),
        pl.BlockSpec((NB, HID), lambda i: (i, 0)),
        pl.BlockSpec((NB, HID), lambda i: (i, 0)),
        full((HC, HID)), full((1, HID)), full((1, HID)), full((1, HID)),
    ]
    if last:
        in_specs += [full((HID, HID)), full((1, HID))]
        out_specs = [pl.BlockSpec((NB, HID), lambda i: (i, 0))]
        out_shape = [jax.ShapeDtypeStruct((N, HID), F32)]
    else:
        in_specs += [full((HID, 8)), full((HID, 8)), full((1, 8)), full((HID, HC))]
        out_specs = [
            pl.BlockSpec((NB, HID), lambda i: (i, 0)),
            pl.BlockSpec((NB, 8), lambda i: (i, 0)),
            pl.BlockSpec((2, NB, 128), lambda i: (0, i, 0)),
            pl.BlockSpec((1, 1, 8), lambda i: (i, 0, 0)),
        ]
        out_shape = [
            jax.ShapeDtypeStruct((N, HID), F32),
            jax.ShapeDtypeStruct((N, 8), F32),
            jax.ShapeDtypeStruct((2, N, 128), F32),
            jax.ShapeDtypeStruct((GRID, 1, 8), F32),
        ]
    return pl.pallas_call(
        functools.partial(_tc_mid_body, last),
        grid=(GRID,),
        in_specs=in_specs,
        out_specs=out_specs,
        out_shape=out_shape,
    )(agg, s, zp, r, wagg, bias, lng, lnb, *rest)


def _sc_body(tab_hbm, xh_hbm, src_hbm, dst_hbm, gmax_hbm,
             agg_out, s_out,
             idxs, idxd, idxg, idxe0, idxe1, srows, drows, xbuf, exb0, exb1,
             idxs_b, idxd_b, idxg_b, idxe0_b, idxe1_b, srows_b, drows_b,
             xbuf_b, exb0_b, exb1_b,
             idxs_t, idxd_t, idxg_t, idxe0_t, idxe1_t, srows_t, drows_t,
             xbuf_t, exb0_t, exb1_t,
             gvec, zb, agg_sh, s_sh, sem, sem_b):
    c = lax.axis_index("c")
    sid = lax.axis_index("s")
    zero16 = jnp.zeros((16,), F32)
    iota16 = lax.iota(I32, 16)

    pltpu.sync_copy(gmax_hbm, gvec)

    @pl.loop(0, 64)
    def _zb(i):
        for j in range(8):
            zb[i, pl.ds(j * 16, 16)] = zero16

    for k in range(10):
        off = sid * ROWS_PER_SUB + k * 64
        pltpu.sync_copy(zb, agg_sh.at[pl.ds(off, 64)])
        pltpu.sync_copy(zb.at[0],
                        s_sh.at[pl.ds(sid * 2 * ROWS_PER_SUB + k * 128, 128)])
    plsc.subcore_barrier()

    gbase = jnp.where(c == 0, 0, N).astype(I32)
    gv = gvec[...]
    g0 = jnp.where(c == 0, gv[0], gv[2])
    g1 = jnp.where(c == 0, gv[1], gv[3])
    cf = c * 2

    def lrelu(x):
        return jnp.maximum(x, x * 0.2)

    def load_and_issue(ebase, chn, b_idxs, b_idxd, b_idxg, b_idxe0, b_idxe1,
                       b_srows, b_drows, b_xbuf, b_sem):
        pltpu.sync_copy(src_hbm.at[pl.ds(ebase, chn)], b_idxs)
        pltpu.sync_copy(dst_hbm.at[pl.ds(ebase, chn)], b_idxd)

        @pl.loop(0, chn // 16)
        def _adj(i):
            sl = pl.ds(i * 16, 16)
            b_idxg[sl] = b_idxs[sl] + gbase
            d2 = b_idxd[sl] * 2
            b_idxe0[sl] = d2
            b_idxe1[sl] = d2 + 1

        pltpu.async_copy(tab_hbm.at[b_idxs], b_srows, b_sem)
        pltpu.async_copy(tab_hbm.at[b_idxd], b_drows, b_sem)
        pltpu.async_copy(xh_hbm.at[b_idxg], b_xbuf, b_sem)

    def wait_gathers(b_idxs, b_idxd, b_idxg, b_srows, b_drows, b_xbuf, b_sem):
        pltpu.make_async_copy(tab_hbm.at[b_idxs], b_srows, b_sem).wait()
        pltpu.make_async_copy(tab_hbm.at[b_idxd], b_drows, b_sem).wait()
        pltpu.make_async_copy(xh_hbm.at[b_idxg], b_xbuf, b_sem).wait()

    def do_chunk(ebase, chn, b_idxs, b_idxd, b_idxg, b_idxe0, b_idxe1,
                 b_srows, b_drows, b_xbuf, b_exb0, b_exb1):
        load_and_issue(ebase, chn, b_idxs, b_idxd, b_idxg, b_idxe0, b_idxe1,
                       b_srows, b_drows, b_xbuf, sem)
        wait_gathers(b_idxs, b_idxd, b_idxg, b_srows, b_drows, b_xbuf, sem)
        compute_scatter(chn, b_idxd, b_idxe0, b_idxe1, b_srows, b_drows,
                        b_xbuf, b_exb0, b_exb1)

    def compute_scatter(chn, b_idxd, b_idxe0, b_idxe1, b_srows, b_drows,
                        b_xbuf, b_exb0, b_exb1):
        @pl.loop(0, chn // 16)
        def _grp(gi):
            rows = gi * 16 + iota16
            s0 = plsc.load_gather(b_srows, [rows, jnp.full((16,), cf, I32)])
            s1 = plsc.load_gather(b_srows, [rows, jnp.full((16,), cf + 1, I32)])
            d0 = plsc.load_gather(b_drows, [rows, jnp.full((16,), cf + 4, I32)])
            d1 = plsc.load_gather(b_drows, [rows, jnp.full((16,), cf + 5, I32)])
            ex0 = jnp.exp(lrelu(s0 + d0) - lrelu(d0 + g0))
            ex1 = jnp.exp(lrelu(s1 + d1) - lrelu(d1 + g1))
            esl = pl.ds(gi * 16, 16)
            b_exb0[esl] = ex0
            b_exb1[esl] = ex1
            for f in range(128):
                colv = jnp.full((16,), f, I32)
                v = plsc.load_gather(b_xbuf, [rows, colv])
                v = v * (ex0 if f < 64 else ex1)
                plsc.store_scatter(b_xbuf, [rows, colv], v)

        pltpu.sync_copy(b_exb0, s_sh.at[b_idxe0], add=True)
        pltpu.sync_copy(b_exb1, s_sh.at[b_idxe1], add=True)
        pltpu.sync_copy(b_xbuf, agg_sh.at[b_idxd], add=True)

    bufA = (idxs, idxd, idxg, idxe0, idxe1, srows, drows, xbuf)
    bufB = (idxs_b, idxd_b, idxg_b, idxe0_b, idxe1_b, srows_b, drows_b, xbuf_b)
    ebase0 = sid * ES
    HN = NCHK // 2
    load_and_issue(ebase0, CHK, *bufA, sem)

    @pl.loop(0, HN)
    def _chunks(t):
        e_a = ebase0 + (2 * t) * CHK
        load_and_issue(e_a + CHK, CHK, *bufB, sem_b)
        wait_gathers(idxs, idxd, idxg, srows, drows, xbuf, sem)
        compute_scatter(CHK, idxd, idxe0, idxe1, srows, drows, xbuf,
                        exb0, exb1)

        @pl.when(t < HN - 1)
        def _():
            load_and_issue(e_a + 2 * CHK, CHK, *bufA, sem)

        wait_gathers(idxs_b, idxd_b, idxg_b, srows_b, drows_b, xbuf_b, sem_b)
        compute_scatter(CHK, idxd_b, idxe0_b, idxe1_b, srows_b, drows_b,
                        xbuf_b, exb0_b, exb1_b)

    if TAIL:
        do_chunk(ebase0 + NCHK * CHK, TAIL,
                 idxs_t, idxd_t, idxg_t, idxe0_t, idxe1_t, srows_t, drows_t,
                 xbuf_t, exb0_t, exb1_t)

    plsc.subcore_barrier()

    rowoff = sid * ROWS_PER_SUB
    for k in range(10):
        off = rowoff + k * 64
        pltpu.sync_copy(agg_sh.at[pl.ds(off, 64)], zb)
        pltpu.sync_copy(zb, agg_out.at[pl.ds(c * NP_ + off, 64)])
    soff = sid * 2 * ROWS_PER_SUB
    for k in range(10):
        pltpu.sync_copy(s_sh.at[pl.ds(soff + k * 128, 128)], zb.at[0])
        pltpu.sync_copy(zb.at[0],
                        s_out.at[pl.ds(c * 2 * NP_ + soff + k * 128, 128)])


_sc_edge = pl.kernel(
    _sc_body,
    out_type=(
        jax.ShapeDtypeStruct((2 * NP_, 128), F32),
        jax.ShapeDtypeStruct((2 * 2 * NP_,), F32),
    ),
    mesh=plsc.VectorSubcoreMesh(core_axis_name="c", subcore_axis_name="s"),
    compiler_params=pltpu.CompilerParams(needs_layout_passes=False,
                                         use_tc_tiling_on_sc=False),
    scratch_types=[
        pltpu.VMEM((CHK,), I32),
        pltpu.VMEM((CHK,), I32),
        pltpu.VMEM((CHK,), I32),
        pltpu.VMEM((CHK,), I32),
        pltpu.VMEM((CHK,), I32),
        pltpu.VMEM((CHK, 8), F32),
        pltpu.VMEM((CHK, 8), F32),
        pltpu.VMEM((CHK, 128), F32),
        pltpu.VMEM((CHK,), F32),
        pltpu.VMEM((CHK,), F32),
        pltpu.VMEM((CHK,), I32),
        pltpu.VMEM((CHK,), I32),
        pltpu.VMEM((CHK,), I32),
        pltpu.VMEM((CHK,), I32),
        pltpu.VMEM((CHK,), I32),
        pltpu.VMEM((CHK, 8), F32),
        pltpu.VMEM((CHK, 8), F32),
        pltpu.VMEM((CHK, 128), F32),
        pltpu.VMEM((CHK,), F32),
        pltpu.VMEM((CHK,), F32),
        pltpu.VMEM((16,), I32),
        pltpu.VMEM((16,), I32),
        pltpu.VMEM((16,), I32),
        pltpu.VMEM((16,), I32),
        pltpu.VMEM((16,), I32),
        pltpu.VMEM((16, 8), F32),
        pltpu.VMEM((16, 8), F32),
        pltpu.VMEM((16, 128), F32),
        pltpu.VMEM((16,), F32),
        pltpu.VMEM((16,), F32),
        pltpu.VMEM((16,), F32),
        pltpu.VMEM((64, 128), F32),
        pltpu.VMEM_SHARED((NP_, 128), F32),
        pltpu.VMEM_SHARED((2 * NP_,), F32),
        pltpu.SemaphoreType.DMA,
        pltpu.SemaphoreType.DMA,
    ],
)


def _fold_layer(p, wg, b_rd):
    att_src = p['att_src'][0]
    att_dst = p['att_dst'][0]
    att_rd = p['att_rd'][0]
    v = jnp.einsum('khc,hc->kh', p['W_rd'].reshape(HID, H, C), att_rd)
    a = wg @ v
    c0 = b_rd @ v
    psrc = jnp.einsum('khc,hc->kh', p['W_x'].reshape(HID, H, C), att_src)
    pdst = jnp.einsum('khc,hc->kh', p['W_x'].reshape(HID, H, C), att_dst)
    psd = jnp.concatenate([psrc, pdst], axis=1)
    aa = jnp.concatenate([-a, a], axis=1)
    cv = jnp.concatenate([jnp.zeros((4,), F32), c0])[None]
    return psd, aa, cv


def _gmax16(smax):
    g = jnp.max(smax[:, 0, :4], axis=0)
    return jnp.concatenate([g, jnp.zeros((12,), F32)])


def kernel(x_seq, x_residue, edge_index, params):
    src = edge_index[0].astype(I32)
    dst = edge_index[1].astype(I32)
    row = lambda b: b[None]
    psd0, aa0, cv0 = _fold_layer(params['convs'][0], params['W_rd'], params['b_rd'])
    psd1, aa1, cv1 = _fold_layer(params['convs'][1], params['W_rd'], params['b_rd'])

    z0, r, tab0, xh0, smax0 = _tc_pre(
        x_seq, x_residue,
        params['W_xs'], row(params['b_xs']),
        row(params['ln_g'][0]), row(params['ln_b'][0]),
        params['W_xr'], row(params['b_xr']),
        psd0, aa0, cv0, params['convs'][0]['W_x'])

    agg0, s0 = _sc_edge(tab0, xh0.reshape(2 * N, 128), src, dst, _gmax16(smax0))

    p0 = params['convs'][0]
    z1, tab1, xh1, smax1 = _tc_mid(
        False, agg0.reshape(2, NP_, 128)[:, :N], s0.reshape(2, NP_, 2)[:, :N], z0, r,
        p0['W_agg'], row(p0['bias']),
        row(params['ln_g'][1]), row(params['ln_b'][1]),
        psd1, aa1, cv1, params['convs'][1]['W_x'])

    agg1, s1 = _sc_edge(tab1, xh1.reshape(2 * N, 128), src, dst, _gmax16(smax1))

    p1 = params['convs'][1]
    (y,) = _tc_mid(
        True, agg1.reshape(2, NP_, 128)[:, :N], s1.reshape(2, NP_, 2)[:, :N], z1, r,
        p1['W_agg'], row(p1['bias']),
        row(params['ln_g'][2]), row(params['ln_b'][2]),
        params['W_out'], row(params['b_out']))
    return y
